# baseline (speedup 1.0000x reference)
import functools
import math

import jax
import jax.numpy as jnp
from jax import lax
from jax.experimental import pallas as pl
from jax.experimental.pallas import tpu as pltpu

MOMENTUM = 0.5

_TB = 16


def _fwd_body(w2_ref, w1_ref, x1_ref, x2_ref, o1_ref, o2_ref):
    w2 = w2_ref[...]
    w1 = w1_ref[...]
    x1 = x1_ref[...]
    x2 = x2_ref[...]
    dn = (((2,), (1,)), ((0,), (0,)))
    l1 = lax.dot_general(w2, x1, dn, preferred_element_type=jnp.float32)
    l2 = lax.dot_general(w1, x2, dn, preferred_element_type=jnp.float32)
    e1 = jnp.exp(l1 - jnp.max(l1, axis=1, keepdims=True))
    o1 = e1 / jnp.sum(e1, axis=1, keepdims=True)
    e2 = jnp.exp(l2 - jnp.max(o1, axis=1, keepdims=True))
    o2 = e2 / jnp.sum(e2, axis=1, keepdims=True)
    o1_ref[...] = o1
    o2_ref[...] = o2


def _forward(w2, w1, x1, x2):
    B, K1, D = w2.shape
    grid = (B // _TB,)
    return pl.pallas_call(
        _fwd_body,
        grid=grid,
        in_specs=[
            pl.BlockSpec((_TB, K1, D), lambda i: (i, 0, 0)),
            pl.BlockSpec((_TB, K1, D), lambda i: (i, 0, 0)),
            pl.BlockSpec((_TB, D), lambda i: (i, 0)),
            pl.BlockSpec((_TB, D), lambda i: (i, 0)),
        ],
        out_specs=[
            pl.BlockSpec((_TB, K1), lambda i: (i, 0)),
            pl.BlockSpec((_TB, K1), lambda i: (i, 0)),
        ],
        out_shape=[
            jax.ShapeDtypeStruct((B, K1), jnp.float32),
            jax.ShapeDtypeStruct((B, K1), jnp.float32),
        ],
    )(w2, w1, x1, x2)


def kernel(x1, x2, memory_x1, memory_x2, index, idx):
    B, D = x1.shape
    K1 = idx.shape[1]
    flat = idx.reshape(-1)
    w2 = jnp.take(memory_x2, flat, axis=0).reshape(B, K1, D)
    w1 = jnp.take(memory_x1, flat, axis=0).reshape(B, K1, D)
    o1, o2 = _forward(w2, w1, x1, x2)

    def _update(mem, x):
        pos = jnp.take(mem, index, axis=0)
        pos = pos * MOMENTUM + x * (1.0 - MOMENTUM)
        norm = jnp.sqrt(jnp.sum(pos ** 2, axis=1, keepdims=True))
        return mem.at[index].set(pos / norm)

    new_memory_x1 = _update(memory_x1, x1)
    new_memory_x2 = _update(memory_x2, x2)
    return (o1[:, :, None], o2[:, :, None], new_memory_x1, new_memory_x2)


# speedup vs baseline: 3.5320x; 3.5320x over previous
import functools
import math

import jax
import jax.numpy as jnp
from jax import lax
from jax.experimental import pallas as pl
from jax.experimental.pallas import tpu as pltpu
from jax.experimental.pallas import tpu_sc as plsc

MOMENTUM = 0.5

_NC, _NS = 2, 16
_NW = _NC * _NS
_CHUNK = 128


def _sc_gather(table_a, table_b, idx_flat):
    R = idx_flat.shape[0]
    N, D = table_a.shape
    per_w = R // _NW
    n_chunks = per_w // _CHUNK
    mesh = plsc.VectorSubcoreMesh(core_axis_name="c", subcore_axis_name="s")

    @functools.partial(
        pl.kernel,
        out_type=[
            jax.ShapeDtypeStruct((R, D), jnp.float32),
            jax.ShapeDtypeStruct((R, D), jnp.float32),
        ],
        mesh=mesh,
        scratch_types=[
            pltpu.VMEM((2, _CHUNK), jnp.int32),
            pltpu.VMEM((2, _CHUNK, D), jnp.float32),
            pltpu.VMEM((2, _CHUNK, D), jnp.float32),
            pltpu.SemaphoreType.DMA((2,)),
            pltpu.SemaphoreType.DMA((2,)),
        ],
    )
    def k(tab_a, tab_b, idxf, out_a, out_b, idx_v, rows_a, rows_b, sem_a, sem_b):
        wid = lax.axis_index("s") * _NC + lax.axis_index("c")
        base = wid * per_w

        def start(c, slot):
            off = base + c * _CHUNK
            pltpu.sync_copy(idxf.at[pl.ds(off, _CHUNK)], idx_v.at[slot])
            pltpu.async_copy(tab_a.at[idx_v.at[slot]], rows_a.at[slot], sem_a.at[slot])
            pltpu.async_copy(tab_b.at[idx_v.at[slot]], rows_b.at[slot], sem_b.at[slot])

        def drain(c, slot):
            off = base + c * _CHUNK
            pltpu.make_async_copy(tab_a.at[idx_v.at[slot]], rows_a.at[slot], sem_a.at[slot]).wait()
            pltpu.sync_copy(rows_a.at[slot], out_a.at[pl.ds(off, _CHUNK)])
            pltpu.make_async_copy(tab_b.at[idx_v.at[slot]], rows_b.at[slot], sem_b.at[slot]).wait()
            pltpu.sync_copy(rows_b.at[slot], out_b.at[pl.ds(off, _CHUNK)])

        start(0, 0)
        if n_chunks > 1:
            start(1, 1)

        def body(t, _):
            c0 = 2 * t
            drain(c0, 0)

            @pl.when(c0 + 2 < n_chunks)
            def _():
                start(c0 + 2, 0)

            drain(c0 + 1, 1)

            @pl.when(c0 + 3 < n_chunks)
            def _():
                start(c0 + 3, 1)

            return _

        lax.fori_loop(0, n_chunks // 2, body, None)

    return k(table_a, table_b, idx_flat)

_TB = 16


def _fwd_body(w2_ref, w1_ref, x1_ref, x2_ref, o1_ref, o2_ref):
    w2 = w2_ref[...]
    w1 = w1_ref[...]
    x1 = x1_ref[...]
    x2 = x2_ref[...]
    dn = (((2,), (1,)), ((0,), (0,)))
    l1 = lax.dot_general(w2, x1, dn, preferred_element_type=jnp.float32)
    l2 = lax.dot_general(w1, x2, dn, preferred_element_type=jnp.float32)
    e1 = jnp.exp(l1 - jnp.max(l1, axis=1, keepdims=True))
    o1 = e1 / jnp.sum(e1, axis=1, keepdims=True)
    e2 = jnp.exp(l2 - jnp.max(o1, axis=1, keepdims=True))
    o2 = e2 / jnp.sum(e2, axis=1, keepdims=True)
    o1_ref[...] = o1
    o2_ref[...] = o2


def _forward(w2, w1, x1, x2):
    B, K1, D = w2.shape
    grid = (B // _TB,)
    return pl.pallas_call(
        _fwd_body,
        grid=grid,
        in_specs=[
            pl.BlockSpec((_TB, K1, D), lambda i: (i, 0, 0)),
            pl.BlockSpec((_TB, K1, D), lambda i: (i, 0, 0)),
            pl.BlockSpec((_TB, D), lambda i: (i, 0)),
            pl.BlockSpec((_TB, D), lambda i: (i, 0)),
        ],
        out_specs=[
            pl.BlockSpec((_TB, K1), lambda i: (i, 0)),
            pl.BlockSpec((_TB, K1), lambda i: (i, 0)),
        ],
        out_shape=[
            jax.ShapeDtypeStruct((B, K1), jnp.float32),
            jax.ShapeDtypeStruct((B, K1), jnp.float32),
        ],
    )(w2, w1, x1, x2)


def kernel(x1, x2, memory_x1, memory_x2, index, idx):
    B, D = x1.shape
    K1 = idx.shape[1]
    flat = idx.reshape(-1)
    w2f, w1f = _sc_gather(memory_x2, memory_x1, flat)
    o1, o2 = _forward(w2f.reshape(B, K1, D), w1f.reshape(B, K1, D), x1, x2)

    def _update(mem, x):
        pos = jnp.take(mem, index, axis=0)
        pos = pos * MOMENTUM + x * (1.0 - MOMENTUM)
        norm = jnp.sqrt(jnp.sum(pos ** 2, axis=1, keepdims=True))
        return mem.at[index].set(pos / norm)

    new_memory_x1 = _update(memory_x1, x1)
    new_memory_x2 = _update(memory_x2, x2)
    return (o1[:, :, None], o2[:, :, None], new_memory_x1, new_memory_x2)
